# Initial kernel scaffold; baseline (speedup 1.0000x reference)
import jax
import jax.numpy as jnp
from jax.experimental import pallas as pl


def kernel(h, x, mask_ligand, batch, params):
    raise NotImplementedError("write your pallas kernel here")


# speedup vs baseline: 11.5970x; 11.5970x over previous
import functools

import jax
import jax.numpy as jnp
import numpy as np
from jax import lax
from jax.experimental import pallas as pl
from jax.experimental.pallas import tpu as pltpu
from jax.experimental.pallas import tpu_sc as plsc

H = 128
NH = 16
HD = H // NH
G = 20
EF = 4
K = 16
R_MAX = 10.0

BN = 200
BE = BN * K
BR = 200

SC_NC = 2
SC_NS = 16
SC_NW = SC_NC * SC_NS
SC_CH = 128
TD = 256

_INV_SQRT_HD = 1.0 / float(np.sqrt(HD))
_GS_SPACING = R_MAX / (G - 1)
_GS_COEFF = float(-0.5 / _GS_SPACING**2)

_OFFSETS = np.linspace(0.0, R_MAX, G, dtype=np.float32).reshape(1, G)
_EFC = np.array([[0.0, 1.0, 0.0, -1.0],
                 [0.0, 0.0, 1.0, -1.0],
                 [1.0, -1.0, -1.0, 1.0],
                 [0.0, 0.0, 0.0, 1.0]], dtype=np.float32)
_R_EXP = np.zeros((EF, EF * G), dtype=np.float32)
_T_EXP = np.zeros((G, EF * G), dtype=np.float32)
for _i in range(EF):
    for _j in range(G):
        _R_EXP[_i, _i * G + _j] = 1.0
        _T_EXP[_j, _i * G + _j] = 1.0
_SH = np.zeros((H, NH), dtype=np.float32)
for _c in range(H):
    _SH[_c, _c // HD] = 1.0
_SHT = _SH.T.copy()


def _dot(a, b):
    return lax.dot_general(a, b, (((1,), (0,)), ((), ())),
                           preferred_element_type=jnp.float32,
                           precision=lax.Precision.HIGHEST)


def _ln_relu(pre, g, be):
    mu = jnp.mean(pre, axis=1, keepdims=True)
    var = jnp.mean((pre - mu) ** 2, axis=1, keepdims=True)
    nh = (pre - mu) / jnp.sqrt(var + 1e-5) * g + be
    return jnp.maximum(nh, 0.0)


def _expand_edges(a, bn):
    return jnp.broadcast_to(a[:, None, :], (bn, K, a.shape[1])).reshape(bn * K, a.shape[1])


def _sigmoid(z):
    return 1.0 / (1.0 + jnp.exp(-z))


def _knn_body(x_blk, bat_blk, xt_full, batc_full, nbr_ref):
    i = pl.program_id(0)
    rows = x_blk[...]
    xt = xt_full[...]
    npad = xt.shape[1]
    rows2 = jnp.sum(rows * rows, axis=1, keepdims=True)
    col2 = jnp.sum(xt * xt, axis=0, keepdims=True)
    d2 = rows2 - 2.0 * _dot(rows, xt) + col2
    cols = lax.broadcasted_iota(jnp.int32, (1, npad), 1)
    rowids = i * BR + lax.broadcasted_iota(jnp.int32, (BR, 1), 0)
    bad = (bat_blk[...] != batc_full[...]) | (cols == rowids)
    inf = jnp.float32(np.inf)
    d2 = jnp.where(bad, inf, d2)
    outs = []
    big = jnp.int32(2**30)
    for _ in range(K):
        mn = jnp.min(d2, axis=1, keepdims=True)
        cand = jnp.where(d2 == mn, cols, big)
        idx = jnp.min(cand, axis=1, keepdims=True)
        outs.append(idx)
        d2 = jnp.where(cols == idx, inf, d2)
    nbr_ref[...] = jnp.concatenate(outs, axis=1)


def _knn(x, batch, n):
    npad = ((n + 1023) // 1024) * 1024
    xt = jnp.concatenate(
        [x.T, jnp.zeros((3, npad - n), jnp.float32)], axis=1)
    batc = jnp.concatenate(
        [batch.reshape(1, n), jnp.full((1, npad - n), -1, batch.dtype)], axis=1)
    grid = n // BR
    return pl.pallas_call(
        _knn_body,
        grid=(grid,),
        in_specs=[
            pl.BlockSpec((BR, 3), lambda i: (i, 0)),
            pl.BlockSpec((BR, 1), lambda i: (i, 0)),
            pl.BlockSpec((3, npad), lambda i: (0, 0)),
            pl.BlockSpec((1, npad), lambda i: (0, 0)),
        ],
        out_specs=pl.BlockSpec((BR, K), lambda i: (i, 0)),
        out_shape=jax.ShapeDtypeStruct((n, K), jnp.int32),
    )(x, batch.reshape(n, 1), xt, batc)


def _sc_gather(table, idx3, ep, d):
    nch = ep // (SC_NW * SC_CH)
    mesh = plsc.VectorSubcoreMesh(core_axis_name="c", subcore_axis_name="s")

    @functools.partial(
        pl.kernel,
        mesh=mesh,
        out_type=jax.ShapeDtypeStruct((ep, d), jnp.float32),
        scratch_types=[
            pltpu.VMEM((nch, SC_CH), jnp.int32),
            pltpu.VMEM((SC_CH, d), jnp.float32),
            pltpu.SemaphoreType.DMA,
        ],
    )
    def gk(table_hbm, idx_hbm, out_hbm, idx_v, rows_v, sem):
        c = lax.axis_index("c")
        s = lax.axis_index("s")
        wid = s * SC_NC + c
        pltpu.sync_copy(idx_hbm.at[wid], idx_v)

        def body(j, carry):
            pltpu.async_copy(table_hbm.at[idx_v.at[j]], rows_v, sem).wait()
            row0 = (wid * nch + j) * SC_CH
            pltpu.sync_copy(rows_v, out_hbm.at[pl.ds(row0, SC_CH)])
            return carry

        lax.fori_loop(0, nch, body, 0)

    return gk(table, idx3)


def _x2h_body(h_blk, x_blk, m_blk, g1_blk,
              efc, rexp, texp, off, sh, sht,
              kv_ef, kv_rf, kv_hd, kv_hs, kv_b1, kv_g, kv_be,
              kw2, kb2, vw2, vb2, ewt, ewb,
              qw1, qb1, qg, qbe, qw2, qb2,
              ow1a, ow1b, ob1, og, obe, ow2, ob2,
              hout_ref):
    h = h_blk[...]
    g1 = g1_blk[...]
    hs = g1[:, 0:H]
    xs = g1[:, H:H + 3]
    ms = g1[:, H + 3:H + 4]

    xd = _expand_edges(x_blk[...], BN)
    md = _expand_edges(m_blk[...], BN)

    rel = xd - xs
    dist = jnp.sqrt(jnp.sum(rel * rel, axis=1, keepdims=True))
    gs = jnp.exp(_GS_COEFF * (dist - off[...]) ** 2)
    ns = (ms == 1.0).astype(jnp.float32)
    nd = (md == 1.0).astype(jnp.float32)
    efm = efc[...]
    ef = ns * efm[0:1] + nd * efm[1:2] + (ns * nd) * efm[2:3] + efm[3:4]
    rf = _dot(ef, rexp[...]) * _dot(gs, texp[...])

    hd_pre = _dot(h, kv_hd[...])
    pre = (_dot(ef, kv_ef[...]) + _dot(rf, kv_rf[...])
           + _dot(hs, kv_hs[...]) + _expand_edges(hd_pre, BN) + kv_b1[...])
    ak = _ln_relu(pre[:, 0:H], kv_g[:, 0:H], kv_be[:, 0:H])
    av = _ln_relu(pre[:, H:2 * H], kv_g[:, H:2 * H], kv_be[:, H:2 * H])
    kk = _dot(ak, kw2[...]) + kb2[...]
    vv = _dot(av, vw2[...]) + vb2[...]
    e_w = _sigmoid(jnp.sum(rf * ewt[...], axis=1, keepdims=True) + ewb[...])
    vv = vv * e_w

    q = _dot(_ln_relu(_dot(h, qw1[...]) + qb1[...], qg[...], qbe[...]),
             qw2[...]) + qb2[...]
    qe = _expand_edges(q, BN)

    scores = _dot(qe * (kk * _INV_SQRT_HD), sh[...])
    s3 = scores.reshape(BN, K, NH)
    smax = jnp.max(s3, axis=1, keepdims=True)
    ex = jnp.exp(s3 - smax)
    den = jnp.sum(ex, axis=1, keepdims=True)
    alpha = (ex / (den + 1e-16)).reshape(BE, NH)
    af = _dot(alpha, sht[...])
    outn = (af * vv).reshape(BN, K, H).sum(axis=1)

    o_pre = _dot(outn, ow1a[...]) + _dot(h, ow1b[...]) + ob1[...]
    o = _dot(_ln_relu(o_pre, og[...], obe[...]), ow2[...]) + ob2[...]
    hout_ref[...] = o + h


def _h2x_body(h_blk, x_blk, m_blk, g2_blk,
              efc, rexp, texp, off, sh,
              kv_ef, kv_rf, kv_hd, kv_hs, kv_b1, kv_g, kv_be,
              kw2, kb2, vw2, vb2, ewt, ewb,
              qw1, qb1, qg, qbe, qw2, qb2,
              xout_ref):
    h = h_blk[...]
    x = x_blk[...]
    mf = m_blk[...]
    g2 = g2_blk[...]
    hs = g2[:, 0:H]
    xs = g2[:, H:H + 3]
    ms = g2[:, H + 3:H + 4]

    xd = _expand_edges(x, BN)
    md = _expand_edges(mf, BN)

    rel = xd - xs
    dist = jnp.sqrt(jnp.sum(rel * rel, axis=1, keepdims=True))
    gs = jnp.exp(_GS_COEFF * (dist - off[...]) ** 2)
    ns = (ms == 1.0).astype(jnp.float32)
    nd = (md == 1.0).astype(jnp.float32)
    efm = efc[...]
    ef = ns * efm[0:1] + nd * efm[1:2] + (ns * nd) * efm[2:3] + efm[3:4]
    rf = _dot(ef, rexp[...]) * _dot(gs, texp[...])

    hd_pre = _dot(h, kv_hd[...])
    pre = (_dot(ef, kv_ef[...]) + _dot(rf, kv_rf[...])
           + _dot(hs, kv_hs[...]) + _expand_edges(hd_pre, BN) + kv_b1[...])
    ak = _ln_relu(pre[:, 0:H], kv_g[:, 0:H], kv_be[:, 0:H])
    av = _ln_relu(pre[:, H:2 * H], kv_g[:, H:2 * H], kv_be[:, H:2 * H])
    kk = _dot(ak, kw2[...]) + kb2[...]
    vv = _dot(av, vw2[...]) + vb2[...]
    e_w = _sigmoid(jnp.sum(rf * ewt[...], axis=1, keepdims=True) + ewb[...])
    vv = vv * e_w

    q = _dot(_ln_relu(_dot(h, qw1[...]) + qb1[...], qg[...], qbe[...]),
             qw2[...]) + qb2[...]
    qe = _expand_edges(q, BN)

    scores = _dot(qe * (kk * _INV_SQRT_HD), sh[...])
    s3 = scores.reshape(BN, K, NH)
    smax = jnp.max(s3, axis=1, keepdims=True)
    ex = jnp.exp(s3 - smax)
    den = jnp.sum(ex, axis=1, keepdims=True)
    alpha = (ex / (den + 1e-16)).reshape(BE, NH)

    w_e = jnp.sum(alpha * vv, axis=1, keepdims=True) * (1.0 / NH)
    delta = (w_e * rel).reshape(BN, K, 3).sum(axis=1)
    xout_ref[...] = x + delta * mf


def _row(v):
    return v.reshape(1, -1)


def _prep_kv(pa, pb):
    w1 = jnp.concatenate([pa['w1'], pb['w1']], axis=1)
    return dict(
        ef=w1[0:EF], rf=w1[EF:EF + EF * G],
        hd=w1[EF + EF * G:EF + EF * G + H], hs=w1[EF + EF * G + H:],
        b1=_row(jnp.concatenate([pa['b1'], pb['b1']])),
        g=_row(jnp.concatenate([pa['g'], pb['g']])),
        be=_row(jnp.concatenate([pa['be'], pb['be']])),
        aw2=pa['w2'], ab2=_row(pa['b2']),
        bw2=pb['w2'], bb2=_row(pb['b2']),
    )


def _prep_mlp(p):
    return (p['w1'], _row(p['b1']), _row(p['g']), _row(p['be']),
            p['w2'], _row(p['b2']))


def _const_ops():
    return (jnp.asarray(_EFC), jnp.asarray(_R_EXP), jnp.asarray(_T_EXP),
            jnp.asarray(_OFFSETS), jnp.asarray(_SH), jnp.asarray(_SHT))


def _full_spec(a):
    return pl.BlockSpec(a.shape, lambda i: tuple(0 for _ in a.shape))


def _x2h_call(h, x, mf, g1, p, n):
    kv = _prep_kv(p['hk'], p['hv'])
    qm = _prep_mlp(p['hq'])
    om = _prep_mlp(p['out'])
    ow1a, ow1b = om[0][0:H], om[0][H:2 * H]
    consts = _const_ops()
    ops = [h, x, mf, g1, *consts,
           kv['ef'], kv['rf'], kv['hd'], kv['hs'], kv['b1'], kv['g'], kv['be'],
           kv['aw2'], kv['ab2'], kv['bw2'], kv['bb2'],
           p['ew_w'].reshape(1, EF * G), p['ew_b'].reshape(1, 1),
           qm[0], qm[1], qm[2], qm[3], qm[4], qm[5],
           ow1a, ow1b, om[1], om[2], om[3], om[4], om[5]]
    in_specs = [
        pl.BlockSpec((BN, H), lambda i: (i, 0)),
        pl.BlockSpec((BN, 3), lambda i: (i, 0)),
        pl.BlockSpec((BN, 1), lambda i: (i, 0)),
        pl.BlockSpec((BE, TD), lambda i: (i, 0)),
    ] + [_full_spec(a) for a in ops[4:]]
    return pl.pallas_call(
        _x2h_body,
        grid=(n // BN,),
        in_specs=in_specs,
        out_specs=pl.BlockSpec((BN, H), lambda i: (i, 0)),
        out_shape=jax.ShapeDtypeStruct((n, H), jnp.float32),
    )(*ops)


def _h2x_call(h, x, mf, g2, p, n):
    kv = _prep_kv(p['xk'], p['xv'])
    qm = _prep_mlp(p['xq'])
    consts = _const_ops()
    ops = [h, x, mf, g2, *consts[:5],
           kv['ef'], kv['rf'], kv['hd'], kv['hs'], kv['b1'], kv['g'], kv['be'],
           kv['aw2'], kv['ab2'], kv['bw2'], kv['bb2'],
           p['ew_w'].reshape(1, EF * G), p['ew_b'].reshape(1, 1),
           qm[0], qm[1], qm[2], qm[3], qm[4], qm[5]]
    in_specs = [
        pl.BlockSpec((BN, H), lambda i: (i, 0)),
        pl.BlockSpec((BN, 3), lambda i: (i, 0)),
        pl.BlockSpec((BN, 1), lambda i: (i, 0)),
        pl.BlockSpec((BE, TD), lambda i: (i, 0)),
    ] + [_full_spec(a) for a in ops[4:]]
    return pl.pallas_call(
        _h2x_body,
        grid=(n // BN,),
        in_specs=in_specs,
        out_specs=pl.BlockSpec((BN, 3), lambda i: (i, 0)),
        out_shape=jax.ShapeDtypeStruct((n, 3), jnp.float32),
    )(*ops)


def kernel(h, x, mask_ligand, batch, params):
    n = h.shape[0]
    e = n * K
    ep = ((e + SC_NW * SC_CH - 1) // (SC_NW * SC_CH)) * (SC_NW * SC_CH)
    h = h.astype(jnp.float32)
    x = x.astype(jnp.float32)
    mf = mask_ligand.astype(jnp.float32).reshape(n, 1)

    nbr = _knn(x, batch.astype(jnp.int32), n)
    src = nbr.reshape(e)
    idx3 = jnp.concatenate(
        [src, jnp.zeros((ep - e,), jnp.int32)]).reshape(SC_NW, ep // (SC_NW * SC_CH), SC_CH)

    pad = jnp.zeros((n, TD - H - 4), jnp.float32)
    for p in params:
        table = jnp.concatenate([h, x, mf, pad], axis=1)
        g1 = _sc_gather(table, idx3, ep, TD)
        h = _x2h_call(h, x, mf, g1, p['x2h'], n)
        table2 = jnp.concatenate([h, x, mf, pad], axis=1)
        g2 = _sc_gather(table2, idx3, ep, TD)
        x = _h2x_call(h, x, mf, g2, p['h2x'], n)
    return (x, h)
